# Initial kernel scaffold; baseline (speedup 1.0000x reference)
import jax
import jax.numpy as jnp
from jax.experimental import pallas as pl


def kernel(hidden_states, W, available_experts):
    raise NotImplementedError("write your pallas kernel here")


# speedup vs baseline: 1.1253x; 1.1253x over previous
import functools

import jax
import jax.numpy as jnp
from jax import lax
from jax.experimental import pallas as pl
from jax.experimental.pallas import tpu as pltpu
from jax.experimental.pallas import tpu_sc as plsc

E = 64
K = 8
H = 4096
T = 8192
T_BLK = 512
N_WORKERS = 32
ROWS_PER_W = T // N_WORKERS
GROUPS = ROWS_PER_W // 16


def _tc_body(x_ref, w_ref, m_ref, logits_ref, maskedT_ref):
    x = x_ref[...]
    w = w_ref[...]
    lt = lax.dot_general(x, w, (((1,), (1,)), ((), ())),
                         preferred_element_type=jnp.float32)
    logits_ref[...] = lt
    maskedT_ref[...] = (lt * m_ref[...]).T


def _tc_router(x, w, mask_row):
    return pl.pallas_call(
        _tc_body,
        grid=(T // T_BLK,),
        in_specs=[
            pl.BlockSpec((T_BLK, H), lambda i: (i, 0)),
            pl.BlockSpec((E, H), lambda i: (0, 0)),
            pl.BlockSpec((1, E), lambda i: (0, 0)),
        ],
        out_specs=[
            pl.BlockSpec((T_BLK, E), lambda i: (i, 0)),
            pl.BlockSpec((E, T_BLK), lambda i: (0, i)),
        ],
        out_shape=[
            jax.ShapeDtypeStruct((T, E), jnp.float32),
            jax.ShapeDtypeStruct((E, T), jnp.float32),
        ],
    )(x, w, mask_row)


@functools.partial(
    pl.kernel,
    mesh=plsc.VectorSubcoreMesh(core_axis_name="c", subcore_axis_name="s"),
    out_type=[
        jax.ShapeDtypeStruct((N_WORKERS, K, ROWS_PER_W), jnp.float32),
        jax.ShapeDtypeStruct((N_WORKERS, K, ROWS_PER_W), jnp.int32),
    ],
    scratch_types=[
        pltpu.VMEM((E, ROWS_PER_W), jnp.float32),
        pltpu.VMEM((K, ROWS_PER_W), jnp.float32),
        pltpu.VMEM((K, ROWS_PER_W), jnp.int32),
    ],
)
def _sc_topk(maskedT_hbm, rw_hbm, se_hbm, ltb, wv, iv):
    wid = lax.axis_index("s") * 2 + lax.axis_index("c")
    base = wid * ROWS_PER_W
    pltpu.sync_copy(maskedT_hbm.at[:, pl.ds(base, ROWS_PER_W)], ltb)

    def group(g, carry):
        col0 = g * 16

        def insert(e, st):
            ts, ix = st
            c = ltb[e, pl.ds(col0, 16)]
            ci = jnp.broadcast_to(e, (16,))
            nts, nix = [], []
            for j in range(K):
                m = c > ts[j]
                nt = jnp.where(m, c, ts[j])
                c = jnp.where(m, ts[j], c)
                ni = jnp.where(m, ci, ix[j])
                ci = jnp.where(m, ix[j], ci)
                nts.append(nt)
                nix.append(ni)
            return (tuple(nts), tuple(nix))

        init = (
            tuple(jnp.full((16,), -jnp.inf, jnp.float32) for _ in range(K)),
            tuple(jnp.zeros((16,), jnp.int32) for _ in range(K)),
        )
        ts, ix = lax.fori_loop(0, E, insert, init)
        inv = 1.0 / (ts[0] + ts[1] + ts[2] + ts[3] + ts[4] + ts[5] + ts[6] + ts[7])
        for j in range(K):
            wv[j, pl.ds(col0, 16)] = ts[j] * inv
            iv[j, pl.ds(col0, 16)] = ix[j]
        return carry

    lax.fori_loop(0, GROUPS, group, 0)
    pltpu.sync_copy(wv, rw_hbm.at[wid])
    pltpu.sync_copy(iv, se_hbm.at[wid])


def kernel(hidden_states, W, available_experts):
    mask_row = (
        jnp.zeros((E,), jnp.float32).at[available_experts].set(1.0).reshape(1, E)
    )
    router_logits, maskedT = _tc_router(hidden_states, W, mask_row)
    rw_kt, se_kt = _sc_topk(maskedT)
    routing_weights = rw_kt.transpose(0, 2, 1).reshape(T, K)
    selected_experts = se_kt.transpose(0, 2, 1).reshape(T, K)
    return (router_logits, routing_weights, selected_experts)
